# baseline (speedup 1.0000x reference)
import functools

import jax
import jax.numpy as jnp
import numpy as np
from jax.experimental import pallas as pl
from jax.experimental.pallas import tpu as pltpu
from jax.experimental.pallas import tpu_sc as plsc

F32 = jnp.float32
I32 = jnp.int32

_B = 2
_L = 2048
_D = 768
_H = 12
_DH = 64
_DT = 8
_DFF = 2048
_NUM_APP = 10000
_NHASH = 4
_BUCKET = 4
_NB = _L // _BUCKET
_NS = _NHASH * _L
_TILE = 256
_NT_L = _L // _TILE
_NT_S = _NS // _TILE
_NPAD = 10240


def _dotT(a, b):
    return jax.lax.dot_general(a, b, (((1,), (1,)), ((), ())),
                               preferred_element_type=F32)


def _dot(a, b):
    return jnp.dot(a, b, preferred_element_type=F32)


def _dotT_hi(a, b):
    return jax.lax.dot_general(a, b, (((1,), (1,)), ((), ())),
                               preferred_element_type=F32,
                               precision=jax.lax.Precision.HIGHEST)


def _dot_hi(a, b):
    return jnp.dot(a, b, preferred_element_type=F32,
                   precision=jax.lax.Precision.HIGHEST)


def _embed_body(idx_ref, xt_ref, wt_ref, bt_ref, app_ref, out_ref):
    xt = xt_ref[0]
    t = _dot(xt, wt_ref[...])
    out_ref[0] = t + bt_ref[...] + app_ref[0]


def _embed(x_app_last, x_time, wt, bt, app_table):
    grid_spec = pltpu.PrefetchScalarGridSpec(
        num_scalar_prefetch=1,
        grid=(_B,),
        in_specs=[
            pl.BlockSpec((1, _L, _DT), lambda b, idx: (b, 0, 0)),
            pl.BlockSpec((_DT, _D), lambda b, idx: (0, 0)),
            pl.BlockSpec((1, _D), lambda b, idx: (0, 0)),
            pl.BlockSpec((1, 1, _D), lambda b, idx: (idx[b], 0, 0)),
        ],
        out_specs=pl.BlockSpec((1, _L, _D), lambda b, idx: (b, 0, 0)),
    )
    return pl.pallas_call(
        _embed_body,
        grid_spec=grid_spec,
        out_shape=jax.ShapeDtypeStruct((_B, _L, _D), F32),
    )(x_app_last, x_time, wt, bt, app_table.reshape(-1, 1, _D))


def _proj_body(x_ref, wqk_ref, wv_ref, it_ref):
    x = x_ref[0]
    qk2 = _dot(x, wqk_ref[...])
    v2 = _dot(x, wv_ref[...])
    it_ref[0, :, :_DH] = qk2[:, :_DH]
    it_ref[0, :, _DH:] = v2[:, :_DH]
    it_ref[1, :, :_DH] = qk2[:, _DH:]
    it_ref[1, :, _DH:] = v2[:, _DH:]


def _proj(x3, wqk, wv):
    return pl.pallas_call(
        _proj_body,
        grid=(_B, _H // 2),
        in_specs=[
            pl.BlockSpec((1, _L, _D), lambda b, h: (b, 0, 0)),
            pl.BlockSpec((_D, 2 * _DH), lambda b, h: (0, h)),
            pl.BlockSpec((_D, 2 * _DH), lambda b, h: (0, h)),
        ],
        out_specs=pl.BlockSpec((2, _L, 2 * _DH),
                               lambda b, h: (b * (_H // 2) + h, 0, 0)),
        out_shape=jax.ShapeDtypeStruct((_B * _H, _L, 2 * _DH), F32),
    )(x3, wqk, wv)


def _hash_body(qk_ref, r_ref, rank_ref, bpc_ref, bpr_ref):
    qk = qk_ref[0][:, :_DH]
    rot = _dot(qk, r_ref[...])
    half = _NB // 2
    lane_half = jax.lax.broadcasted_iota(I32, (_L, _NB), 1)
    pos = jax.lax.broadcasted_iota(I32, (_L, 1), 0)
    ones1 = jnp.ones((1, 1), F32)
    ones_l = jnp.ones((_L, 1), F32)
    for r in range(_NHASH):
        sec = rot[:, r * half:(r + 1) * half]
        full = jnp.concatenate([sec, -sec], axis=1)
        mx = jnp.max(full, axis=1, keepdims=True)
        bucket = jnp.min(jnp.where(full == mx, lane_half, _NB),
                         axis=1, keepdims=True)
        bf_col = bucket.astype(F32)
        brow = _dotT_hi(ones1, bf_col)
        qrow = jax.lax.broadcasted_iota(I32, (_TILE, _L), 1)
        tiles = []
        for t in range(_NT_L):
            bcol = bf_col[t * _TILE:(t + 1) * _TILE]
            pcol = pos[t * _TILE:(t + 1) * _TILE]
            less = brow < bcol - 0.5
            eq = jnp.abs(brow - bcol) < 0.5
            cmp = (less | (eq & (qrow < pcol))).astype(F32)
            tiles.append(_dot(cmp, ones_l))
        rank = jnp.concatenate(tiles, axis=0).astype(I32)
        rank_ref[0, :, r:r + 1] = rank + (pl.program_id(0) * _NS + r * _L)
        for j in range(_BUCKET):
            for jj, slot in ((j, j), (j + _BUCKET, _L - _BUCKET + j)):
                sel = (rank == slot).astype(I32)
                val = jnp.sum(pos * sel, axis=0, keepdims=True)
                bpc_ref[0, jj:jj + 1, r:r + 1] = val
                bpr_ref[0, r:r + 1, jj:jj + 1] = val


def _hash(items, r_mat):
    bh = _B * _H
    return pl.pallas_call(
        _hash_body,
        grid=(bh,),
        in_specs=[
            pl.BlockSpec((1, _L, 2 * _DH), lambda i: (i, 0, 0)),
            pl.BlockSpec((_DH, _NHASH * (_NB // 2)), lambda i: (0, 0)),
        ],
        out_specs=[
            pl.BlockSpec((1, _L, _NHASH), lambda i: (i, 0, 0)),
            pl.BlockSpec((1, 2 * _BUCKET, _NHASH), lambda i: (i, 0, 0)),
            pl.BlockSpec((1, _NHASH, 2 * _BUCKET), lambda i: (i, 0, 0)),
        ],
        out_shape=[
            jax.ShapeDtypeStruct((bh, _L, _NHASH), I32),
            jax.ShapeDtypeStruct((bh, 2 * _BUCKET, _NHASH), I32),
            jax.ShapeDtypeStruct((bh, _NHASH, 2 * _BUCKET), I32),
        ],
    )(items, r_mat)


_W_ITEM = 128
_W_OUT = 128
_CHUNK = 128
_NWORK = 32


def _sc_scatter_body(items_ref, gd_ref, sorted_ref, src_v, idx_v, sem):
    wid = jax.lax.axis_index("s") * 2 + jax.lax.axis_index("c")
    njob = (_B * _H) * (_L // _CHUNK) // _NWORK
    for j in range(njob):
        g = wid * njob + j
        bh = g // (_L // _CHUNK)
        t = g % (_L // _CHUNK)
        pltpu.sync_copy(
            items_ref.at[pl.ds(bh * _L + t * _CHUNK, _CHUNK), :], src_v)
        for r in range(_NHASH):
            off = bh * (_NHASH * _L) + r * _L + t * _CHUNK
            pltpu.sync_copy(gd_ref.at[pl.ds(off, _CHUNK)], idx_v)
            pltpu.async_copy(src_v, sorted_ref.at[idx_v], sem).wait()


def _sc_scatter(items, gd_flat):
    mesh = plsc.VectorSubcoreMesh(core_axis_name="c", subcore_axis_name="s")
    fn = pl.kernel(
        _sc_scatter_body,
        out_type=jax.ShapeDtypeStruct((_B * _H * _NS, _W_ITEM), F32),
        mesh=mesh,
        scratch_types=[
            pltpu.VMEM((_CHUNK, _W_ITEM), F32),
            pltpu.VMEM((_CHUNK,), I32),
            pltpu.SemaphoreType.DMA,
        ],
    )
    return fn(items, gd_flat)


def _sc_gather_body(sout_ref, gd_ref, out_ref, idx_v, rows_v, sem):
    wid = jax.lax.axis_index("s") * 2 + jax.lax.axis_index("c")
    njob = (_B * _H) * _NS // _CHUNK // _NWORK

    def body(i, _):
        g = wid * njob + i
        pltpu.sync_copy(gd_ref.at[pl.ds(g * _CHUNK, _CHUNK)], idx_v)
        pltpu.async_copy(sout_ref.at[idx_v], rows_v, sem).wait()
        pltpu.sync_copy(rows_v, out_ref.at[pl.ds(g * _CHUNK, _CHUNK), :])
        return 0

    jax.lax.fori_loop(0, njob, body, 0)


def _sc_gather(sout, gd_flat):
    mesh = plsc.VectorSubcoreMesh(core_axis_name="c", subcore_axis_name="s")
    fn = pl.kernel(
        _sc_gather_body,
        out_type=jax.ShapeDtypeStruct((_B * _H * _NS, _W_OUT), F32),
        mesh=mesh,
        scratch_types=[
            pltpu.VMEM((_CHUNK,), I32),
            pltpu.VMEM((_CHUNK, _W_OUT), F32),
            pltpu.SemaphoreType.DMA,
        ],
    )
    return fn(sout, gd_flat)


def _attend_body(srt_ref, bpc_ref, bpr_ref, out_ref, roll_ref):
    roll_ref[0:_BUCKET, :] = srt_ref[0, _NS - _BUCKET:_NS, :]
    roll_ref[_BUCKET:_NS, :] = srt_ref[0, 0:_NS - _BUCKET, :]
    i4 = jax.lax.broadcasted_iota(I32, (_TILE, _TILE), 0) // _BUCKET
    j4 = jax.lax.broadcasted_iota(I32, (_TILE, _TILE), 1) // _BUCKET
    m4 = i4 == j4
    diag = (jax.lax.broadcasted_iota(I32, (_TILE, _TILE), 0)
            == jax.lax.broadcasted_iota(I32, (_TILE, _TILE), 1))
    ones1 = jnp.ones((1, 1), F32)
    ones_d = jnp.ones((_DH, 1), F32)
    ones_c = jnp.ones((_TILE, 1), F32)
    for t in range(_NT_S):
        s0 = _TILE * t
        kv = srt_ref[0, s0:s0 + _TILE, :]
        rkv = roll_ref[s0:s0 + _TILE, :]
        q, v_s = kv[:, :_DH], kv[:, _DH:2 * _DH]
        qp, v_p = rkv[:, :_DH], rkv[:, _DH:2 * _DH]
        nc_s = jnp.sqrt(_dot_hi(q * q, ones_d)) + 1e-9
        nc_p = jnp.sqrt(_dot_hi(qp * qp, ones_d)) + 1e-9
        sc_s = 0.125 / _dotT_hi(ones1, nc_s)
        sc_p = 0.125 / _dotT_hi(ones1, nc_p)
        ds = _dotT(q, q) * sc_s
        dp = _dotT(q, qp) * sc_p
        ds = jnp.where(diag, ds - 1e5, ds)
        if t % (_NT_S // _NHASH) == 0:
            r = t // (_NT_S // _NHASH)
            rp = (r - 1) % _NHASH
            qpos = bpc_ref[0, 0:_BUCKET, r:r + 1]
            kpos = bpr_ref[0, rp:rp + 1, _BUCKET:2 * _BUCKET]
            m44 = (qpos == kpos).astype(I32)
            mbig = jnp.pad(m44, ((0, _TILE - _BUCKET),
                                 (0, _TILE - _BUCKET)))
            dp = jnp.where(mbig == 1, dp - 1e5, dp)
        ds = jnp.where(m4, ds, -1e30)
        dp = jnp.where(m4, dp, -1e30)
        m = jnp.max(jnp.maximum(ds, dp), axis=1, keepdims=True)
        es = jnp.exp(ds - m)
        ep = jnp.exp(dp - m)
        acc = (_dot(es, jnp.concatenate([v_s, ones_c], axis=1))
               + _dot(ep, jnp.concatenate([v_p, ones_c], axis=1)))
        den = acc[:, _DH:_DH + 1]
        out_ref[0, s0:s0 + _TILE, :_DH] = acc[:, :_DH] / den
        out_ref[0, s0:s0 + _TILE, _DH:_DH + 1] = m + jnp.log(den)


def _attend(sorted4, bpc, bpr):
    bh = _B * _H
    return pl.pallas_call(
        _attend_body,
        grid=(bh,),
        in_specs=[
            pl.BlockSpec((1, _NS, _W_ITEM), lambda i: (i, 0, 0)),
            pl.BlockSpec((1, 2 * _BUCKET, _NHASH), lambda i: (i, 0, 0)),
            pl.BlockSpec((1, _NHASH, 2 * _BUCKET), lambda i: (i, 0, 0)),
        ],
        out_specs=pl.BlockSpec((1, _NS, _W_OUT), lambda i: (i, 0, 0)),
        out_shape=jax.ShapeDtypeStruct((bh, _NS, _W_OUT), F32),
        scratch_shapes=[pltpu.VMEM((_NS, _W_ITEM), F32)],
    )(sorted4, bpc, bpr)


def _combine_body(g_ref, o_ref):
    for hh in range(2):
        os_ = [g_ref[0, hh * _NHASH + r, :, :_DH] for r in range(_NHASH)]
        ls_ = [g_ref[0, hh * _NHASH + r, :, _DH:_DH + 1]
               for r in range(_NHASH)]
        m = jnp.maximum(jnp.maximum(ls_[0], ls_[1]),
                        jnp.maximum(ls_[2], ls_[3]))
        ws = [jnp.exp(l - m) for l in ls_]
        den = ws[0] + ws[1] + ws[2] + ws[3]
        o_ref[0, :, hh * _DH:(hh + 1) * _DH] = \
            (os_[0] * ws[0] + os_[1] * ws[1]
             + os_[2] * ws[2] + os_[3] * ws[3]) / den


def _combine(gat):
    gat6 = gat.reshape(_B * _H // 2, 2 * _NHASH, _L, _W_OUT)
    np_ = _B * _H // 2
    return pl.pallas_call(
        _combine_body,
        grid=(np_,),
        in_specs=[pl.BlockSpec((1, 2 * _NHASH, _L, _W_OUT),
                               lambda i: (i, 0, 0, 0))],
        out_specs=pl.BlockSpec((1, _L, 2 * _DH),
                               lambda i: (i // (_H // 2), 0, i % (_H // 2))),
        out_shape=jax.ShapeDtypeStruct((_B, _L, _D), F32),
    )(gat6)


def _ln(x, g, b):
    m = jnp.mean(x, axis=-1, keepdims=True)
    v = jnp.mean((x - m) * (x - m), axis=-1, keepdims=True)
    return (x - m) * jax.lax.rsqrt(v + 1e-5) * g + b


def _post_body(x_ref, o_ref, wo_ref, bo_ref, g1_ref, b1_ref,
               w1_ref, bf1_ref, w2_ref, bf2_ref, g2_ref, b2_ref, out_ref):
    x = x_ref[...]
    y = x + _dot(o_ref[...], wo_ref[...]) + bo_ref[...]
    h1 = _ln(y, g1_ref[...], b1_ref[...])
    a = jax.nn.gelu(_dot(h1, w1_ref[...]) + bf1_ref[...])
    y2 = h1 + _dot(a, w2_ref[...]) + bf2_ref[...]
    out_ref[...] = _ln(y2, g2_ref[...], b2_ref[...])


def _post(x2, o2, p):
    n = _B * _L
    blk = 512
    vec = lambda a: a.reshape(1, -1)
    return pl.pallas_call(
        _post_body,
        grid=(n // blk,),
        in_specs=[
            pl.BlockSpec((blk, _D), lambda i: (i, 0)),
            pl.BlockSpec((blk, _D), lambda i: (i, 0)),
            pl.BlockSpec((_D, _D), lambda i: (0, 0)),
            pl.BlockSpec((1, _D), lambda i: (0, 0)),
            pl.BlockSpec((1, _D), lambda i: (0, 0)),
            pl.BlockSpec((1, _D), lambda i: (0, 0)),
            pl.BlockSpec((_D, _DFF), lambda i: (0, 0)),
            pl.BlockSpec((1, _DFF), lambda i: (0, 0)),
            pl.BlockSpec((_DFF, _D), lambda i: (0, 0)),
            pl.BlockSpec((1, _D), lambda i: (0, 0)),
            pl.BlockSpec((1, _D), lambda i: (0, 0)),
            pl.BlockSpec((1, _D), lambda i: (0, 0)),
        ],
        out_specs=pl.BlockSpec((blk, _D), lambda i: (i, 0)),
        out_shape=jax.ShapeDtypeStruct((n, _D), F32),
    )(x2, o2, p['Wo'], vec(p['bo']), vec(p['g1']), vec(p['b1']),
      p['W1'], vec(p['bf1']), p['W2'], vec(p['bf2']),
      vec(p['g2']), vec(p['b2']))


def _head_body(tgt_ref, xl_ref, tv_ref, gf_ref, bf_ref,
               wp_ref, bp_ref, out_ref):
    x = _ln(xl_ref[...], gf_ref[...], bf_ref[...])
    feat = jnp.concatenate([tv_ref[...], x], axis=1)
    score = _dot(feat, wp_ref[...]) + bp_ref[...]
    m = jnp.max(score, axis=1, keepdims=True)
    lse = m + jnp.log(jnp.sum(jnp.exp(score - m), axis=1, keepdims=True))
    row = jax.lax.broadcasted_iota(I32, (_B, 1), 0)
    tsel = jnp.where(row == 0, tgt_ref[0], tgt_ref[1])
    lane = jax.lax.broadcasted_iota(I32, (_B, _NPAD), 1)
    oh = (lane == tsel).astype(F32)
    st = jnp.sum(score * oh, axis=1, keepdims=True)
    out_ref[...] = -jnp.sum(st - lse, axis=0, keepdims=True) / _B


def _head(targets, x_last, tv_last, gf, bf, wp_pad, bp_pad):
    tvd = tv_last.shape[1]
    grid_spec = pltpu.PrefetchScalarGridSpec(
        num_scalar_prefetch=1,
        grid=(1,),
        in_specs=[
            pl.BlockSpec((_B, _D), lambda i, t: (0, 0)),
            pl.BlockSpec((_B, tvd), lambda i, t: (0, 0)),
            pl.BlockSpec((1, _D), lambda i, t: (0, 0)),
            pl.BlockSpec((1, _D), lambda i, t: (0, 0)),
            pl.BlockSpec((_D + tvd, _NPAD), lambda i, t: (0, 0)),
            pl.BlockSpec((1, _NPAD), lambda i, t: (0, 0)),
        ],
        out_specs=pl.BlockSpec((1, 1), lambda i, t: (0, 0)),
    )
    return pl.pallas_call(
        _head_body,
        grid_spec=grid_spec,
        out_shape=jax.ShapeDtypeStruct((1, 1), F32),
    )(targets, x_last, tv_last, gf, bf, wp_pad, bp_pad)


def kernel(x_app, x_time, time_vec, targets, mode, params):
    del mode
    vec = lambda a: a.reshape(1, -1)
    x = _embed(x_app[:, -1].astype(I32), x_time,
               params['Wt'], vec(params['bt']), params['app_table'])
    x2 = x.reshape(_B * _L, _D)
    for i, p in enumerate(params['layers']):
        items = _proj(x2.reshape(_B, _L, _D), p['Wqk'], p['Wv'])
        r_mat = jax.random.normal(
            jax.random.fold_in(jax.random.key(42), i),
            (_DH, _NHASH, _NB // 2), dtype=F32).reshape(_DH, -1)
        gd, bpc, bpr = _hash(items, r_mat)
        gd_flat = gd.transpose(0, 2, 1).reshape(-1)
        sorted_all = _sc_scatter(items.reshape(_B * _H * _L, _W_ITEM),
                                 gd_flat)
        sout = _attend(sorted_all.reshape(_B * _H, _NS, _W_ITEM), bpc, bpr)
        gat = _sc_gather(sout.reshape(_B * _H * _NS, _W_OUT), gd_flat)
        o2 = _combine(gat).reshape(_B * _L, _D)
        x2 = _post(x2, o2, p)
    x_last = x2.reshape(_B, _L, _D)[:, -1, :]
    tvd = time_vec.shape[-1]
    wp_pad = jnp.pad(params['Wp'], ((0, 0), (0, _NPAD - _NUM_APP)))
    bp_pad = jnp.pad(params['bp'], (0, _NPAD - _NUM_APP),
                     constant_values=-1e30)
    loss = _head(targets.astype(I32), x_last, time_vec[:, -1, :],
                 vec(params['gf']), vec(params['bf']), wp_pad, vec(bp_pad))
    return loss[0, 0]


# speedup vs baseline: 1.4553x; 1.4553x over previous
import functools

import jax
import jax.numpy as jnp
import numpy as np
from jax.experimental import pallas as pl
from jax.experimental.pallas import tpu as pltpu
from jax.experimental.pallas import tpu_sc as plsc

F32 = jnp.float32
I32 = jnp.int32

_B = 2
_L = 2048
_D = 768
_H = 12
_DH = 64
_DT = 8
_DFF = 2048
_NUM_APP = 10000
_NHASH = 4
_BUCKET = 4
_NB = _L // _BUCKET
_NS = _NHASH * _L
_TILE = 256
_NT_L = _L // _TILE
_NT_S = _NS // _TILE
_NPAD = 10240


def _dotT(a, b):
    return jax.lax.dot_general(a, b, (((1,), (1,)), ((), ())),
                               preferred_element_type=F32)


def _dot(a, b):
    return jnp.dot(a, b, preferred_element_type=F32)


def _dotT_hi(a, b):
    return jax.lax.dot_general(a, b, (((1,), (1,)), ((), ())),
                               preferred_element_type=F32,
                               precision=jax.lax.Precision.HIGHEST)


def _dot_hi(a, b):
    return jnp.dot(a, b, preferred_element_type=F32,
                   precision=jax.lax.Precision.HIGHEST)


def _embed_body(idx_ref, xt_ref, wt_ref, bt_ref, app_ref, out_ref):
    xt = xt_ref[0]
    t = _dot(xt, wt_ref[...])
    out_ref[0] = t + bt_ref[...] + app_ref[0]


def _embed(x_app_last, x_time, wt, bt, app_table):
    grid_spec = pltpu.PrefetchScalarGridSpec(
        num_scalar_prefetch=1,
        grid=(_B,),
        in_specs=[
            pl.BlockSpec((1, _L, _DT), lambda b, idx: (b, 0, 0)),
            pl.BlockSpec((_DT, _D), lambda b, idx: (0, 0)),
            pl.BlockSpec((1, _D), lambda b, idx: (0, 0)),
            pl.BlockSpec((1, 1, _D), lambda b, idx: (idx[b], 0, 0)),
        ],
        out_specs=pl.BlockSpec((1, _L, _D), lambda b, idx: (b, 0, 0)),
    )
    return pl.pallas_call(
        _embed_body,
        grid_spec=grid_spec,
        out_shape=jax.ShapeDtypeStruct((_B, _L, _D), F32),
    )(x_app_last, x_time, wt, bt, app_table.reshape(-1, 1, _D))


def _proj_body(x_ref, wqk_ref, wv_ref, it_ref):
    x = x_ref[0]
    qk2 = _dot(x, wqk_ref[...])
    v2 = _dot(x, wv_ref[...])
    it_ref[0, :, :_DH] = qk2[:, :_DH]
    it_ref[0, :, _DH:] = v2[:, :_DH]
    it_ref[1, :, :_DH] = qk2[:, _DH:]
    it_ref[1, :, _DH:] = v2[:, _DH:]


def _proj(x3, wqk, wv):
    return pl.pallas_call(
        _proj_body,
        grid=(_B, _H // 2),
        in_specs=[
            pl.BlockSpec((1, _L, _D), lambda b, h: (b, 0, 0)),
            pl.BlockSpec((_D, 2 * _DH), lambda b, h: (0, h)),
            pl.BlockSpec((_D, 2 * _DH), lambda b, h: (0, h)),
        ],
        out_specs=pl.BlockSpec((2, _L, 2 * _DH),
                               lambda b, h: (b * (_H // 2) + h, 0, 0)),
        out_shape=jax.ShapeDtypeStruct((_B * _H, _L, 2 * _DH), F32),
    )(x3, wqk, wv)


def _hash_body(qk_ref, r_ref, rank_ref, bpc_ref, bpr_ref):
    qk = qk_ref[0][:, :_DH]
    rot = _dot(qk, r_ref[...])
    half = _NB // 2
    lane_half = jax.lax.broadcasted_iota(I32, (_L, _NB), 1)
    pos = jax.lax.broadcasted_iota(I32, (_L, 1), 0)
    ones1 = jnp.ones((1, 1), F32)
    ones_l = jnp.ones((_L, 1), F32)
    for r in range(_NHASH):
        sec = rot[:, r * half:(r + 1) * half]
        full = jnp.concatenate([sec, -sec], axis=1)
        mx = jnp.max(full, axis=1, keepdims=True)
        bucket = jnp.min(jnp.where(full == mx, lane_half, _NB),
                         axis=1, keepdims=True)
        bf_col = bucket.astype(F32)
        brow = _dotT_hi(ones1, bf_col)
        qrow = jax.lax.broadcasted_iota(I32, (_TILE, _L), 1)
        tiles = []
        for t in range(_NT_L):
            bcol = bf_col[t * _TILE:(t + 1) * _TILE]
            pcol = pos[t * _TILE:(t + 1) * _TILE]
            less = brow < bcol - 0.5
            eq = jnp.abs(brow - bcol) < 0.5
            cmp = (less | (eq & (qrow < pcol))).astype(F32)
            tiles.append(_dot(cmp, ones_l))
        rank = jnp.concatenate(tiles, axis=0).astype(I32)
        rank_ref[0, :, r:r + 1] = rank + (pl.program_id(0) * _NS + r * _L)
        for j in range(_BUCKET):
            for jj, slot in ((j, j), (j + _BUCKET, _L - _BUCKET + j)):
                sel = (rank == slot).astype(I32)
                val = jnp.sum(pos * sel, axis=0, keepdims=True)
                bpc_ref[0, jj:jj + 1, r:r + 1] = val
                bpr_ref[0, r:r + 1, jj:jj + 1] = val


def _hash(items, r_mat):
    bh = _B * _H
    return pl.pallas_call(
        _hash_body,
        grid=(bh,),
        in_specs=[
            pl.BlockSpec((1, _L, 2 * _DH), lambda i: (i, 0, 0)),
            pl.BlockSpec((_DH, _NHASH * (_NB // 2)), lambda i: (0, 0)),
        ],
        out_specs=[
            pl.BlockSpec((1, _L, _NHASH), lambda i: (i, 0, 0)),
            pl.BlockSpec((1, 2 * _BUCKET, _NHASH), lambda i: (i, 0, 0)),
            pl.BlockSpec((1, _NHASH, 2 * _BUCKET), lambda i: (i, 0, 0)),
        ],
        out_shape=[
            jax.ShapeDtypeStruct((bh, _L, _NHASH), I32),
            jax.ShapeDtypeStruct((bh, 2 * _BUCKET, _NHASH), I32),
            jax.ShapeDtypeStruct((bh, _NHASH, 2 * _BUCKET), I32),
        ],
    )(items, r_mat)


_W_ITEM = 128
_W_OUT = 128
_CHUNK = 128
_NWORK = 32


def _sc_scatter_body(items_ref, gd_ref, sorted_ref, src_v, idx_v, sem):
    wid = jax.lax.axis_index("s") * 2 + jax.lax.axis_index("c")
    njob = (_B * _H) * (_L // _CHUNK) // _NWORK
    for j in range(njob):
        g = wid * njob + j
        bh = g // (_L // _CHUNK)
        t = g % (_L // _CHUNK)
        pltpu.sync_copy(
            items_ref.at[pl.ds(bh * _L + t * _CHUNK, _CHUNK), :], src_v)
        for r in range(_NHASH):
            off = bh * (_NHASH * _L) + r * _L + t * _CHUNK
            pltpu.sync_copy(gd_ref.at[pl.ds(off, _CHUNK)], idx_v)
            pltpu.async_copy(src_v, sorted_ref.at[idx_v], sem).wait()


def _sc_scatter(items, gd_flat):
    mesh = plsc.VectorSubcoreMesh(core_axis_name="c", subcore_axis_name="s")
    fn = pl.kernel(
        _sc_scatter_body,
        out_type=jax.ShapeDtypeStruct((_B * _H * _NS, _W_ITEM), F32),
        mesh=mesh,
        scratch_types=[
            pltpu.VMEM((_CHUNK, _W_ITEM), F32),
            pltpu.VMEM((_CHUNK,), I32),
            pltpu.SemaphoreType.DMA,
        ],
    )
    return fn(items, gd_flat)


def _sc_gather_body(sout_ref, gd_ref, out_ref, idx_v, rows_v, sem):
    wid = jax.lax.axis_index("s") * 2 + jax.lax.axis_index("c")
    njob = (_B * _H) * _NS // _CHUNK // _NWORK

    def body(i, _):
        g = wid * njob + i
        pltpu.sync_copy(gd_ref.at[pl.ds(g * _CHUNK, _CHUNK)], idx_v)
        pltpu.async_copy(sout_ref.at[idx_v], rows_v, sem).wait()
        pltpu.sync_copy(rows_v, out_ref.at[pl.ds(g * _CHUNK, _CHUNK), :])
        return 0

    jax.lax.fori_loop(0, njob, body, 0)


def _sc_gather(sout, gd_flat):
    mesh = plsc.VectorSubcoreMesh(core_axis_name="c", subcore_axis_name="s")
    fn = pl.kernel(
        _sc_gather_body,
        out_type=jax.ShapeDtypeStruct((_B * _H * _NS, _W_OUT), F32),
        mesh=mesh,
        scratch_types=[
            pltpu.VMEM((_CHUNK,), I32),
            pltpu.VMEM((_CHUNK, _W_OUT), F32),
            pltpu.SemaphoreType.DMA,
        ],
    )
    return fn(sout, gd_flat)


def _attend_body(srt_ref, bpc_ref, bpr_ref, out_ref, knv_ref, roll_ref):
    for t in range(_NT_S):
        s0 = _TILE * t
        kv = srt_ref[0, s0:s0 + _TILE, :]
        q = kv[:, :_DH]
        n = jnp.sqrt(jnp.sum(q * q, axis=1, keepdims=True)) + 1e-9
        knv_ref[s0:s0 + _TILE, :_DH] = q / n
        knv_ref[s0:s0 + _TILE, _DH:] = kv[:, _DH:]
    roll_ref[0:_BUCKET, :] = knv_ref[_NS - _BUCKET:_NS, :]
    roll_ref[_BUCKET:_NS, :] = knv_ref[0:_NS - _BUCKET, :]
    i4 = jax.lax.broadcasted_iota(I32, (_TILE, _TILE), 0) // _BUCKET
    j4 = jax.lax.broadcasted_iota(I32, (_TILE, _TILE), 1) // _BUCKET
    m4 = i4 == j4
    diag = (jax.lax.broadcasted_iota(I32, (_TILE, _TILE), 0)
            == jax.lax.broadcasted_iota(I32, (_TILE, _TILE), 1))
    ones_c = jnp.ones((_TILE, 1), F32)
    for t in range(_NT_S):
        s0 = _TILE * t
        q = srt_ref[0, s0:s0 + _TILE, :_DH]
        kv = knv_ref[s0:s0 + _TILE, :]
        rkv = roll_ref[s0:s0 + _TILE, :]
        kn_s, v_s = kv[:, :_DH], kv[:, _DH:2 * _DH]
        kn_p, v_p = rkv[:, :_DH], rkv[:, _DH:2 * _DH]
        ds = _dotT(q, kn_s) * 0.125
        dp = _dotT(q, kn_p) * 0.125
        ds = jnp.where(diag, ds - 1e5, ds)
        if t % (_NT_S // _NHASH) == 0:
            r = t // (_NT_S // _NHASH)
            rp = (r - 1) % _NHASH
            qpos = bpc_ref[0, 0:_BUCKET, r:r + 1]
            kpos = bpr_ref[0, rp:rp + 1, _BUCKET:2 * _BUCKET]
            m44 = (qpos == kpos).astype(I32)
            mbig = jnp.pad(m44, ((0, _TILE - _BUCKET),
                                 (0, _TILE - _BUCKET)))
            dp = jnp.where(mbig == 1, dp - 1e5, dp)
        ds = jnp.where(m4, ds, -1e30)
        dp = jnp.where(m4, dp, -1e30)
        m = jnp.max(jnp.maximum(ds, dp), axis=1, keepdims=True)
        es = jnp.exp(ds - m)
        ep = jnp.exp(dp - m)
        acc = (_dot(es, jnp.concatenate([v_s, ones_c], axis=1))
               + _dot(ep, jnp.concatenate([v_p, ones_c], axis=1)))
        den = acc[:, _DH:_DH + 1]
        out_ref[0, s0:s0 + _TILE, :_DH] = acc[:, :_DH] / den
        out_ref[0, s0:s0 + _TILE, _DH:_DH + 1] = m + jnp.log(den)


def _attend(sorted4, bpc, bpr):
    bh = _B * _H
    return pl.pallas_call(
        _attend_body,
        grid=(bh,),
        in_specs=[
            pl.BlockSpec((1, _NS, _W_ITEM), lambda i: (i, 0, 0)),
            pl.BlockSpec((1, 2 * _BUCKET, _NHASH), lambda i: (i, 0, 0)),
            pl.BlockSpec((1, _NHASH, 2 * _BUCKET), lambda i: (i, 0, 0)),
        ],
        out_specs=pl.BlockSpec((1, _NS, _W_OUT), lambda i: (i, 0, 0)),
        out_shape=jax.ShapeDtypeStruct((bh, _NS, _W_OUT), F32),
        scratch_shapes=[
            pltpu.VMEM((_NS, _W_ITEM), F32),
            pltpu.VMEM((_NS, _W_ITEM), F32),
        ],
    )(sorted4, bpc, bpr)


def _combine_body(g_ref, o_ref):
    for hh in range(2):
        os_ = [g_ref[0, hh * _NHASH + r, :, :_DH] for r in range(_NHASH)]
        ls_ = [g_ref[0, hh * _NHASH + r, :, _DH:_DH + 1]
               for r in range(_NHASH)]
        m = jnp.maximum(jnp.maximum(ls_[0], ls_[1]),
                        jnp.maximum(ls_[2], ls_[3]))
        ws = [jnp.exp(l - m) for l in ls_]
        den = ws[0] + ws[1] + ws[2] + ws[3]
        o_ref[0, :, hh * _DH:(hh + 1) * _DH] = \
            (os_[0] * ws[0] + os_[1] * ws[1]
             + os_[2] * ws[2] + os_[3] * ws[3]) / den


def _combine(gat):
    gat6 = gat.reshape(_B * _H // 2, 2 * _NHASH, _L, _W_OUT)
    np_ = _B * _H // 2
    return pl.pallas_call(
        _combine_body,
        grid=(np_,),
        in_specs=[pl.BlockSpec((1, 2 * _NHASH, _L, _W_OUT),
                               lambda i: (i, 0, 0, 0))],
        out_specs=pl.BlockSpec((1, _L, 2 * _DH),
                               lambda i: (i // (_H // 2), 0, i % (_H // 2))),
        out_shape=jax.ShapeDtypeStruct((_B, _L, _D), F32),
    )(gat6)


def _ln(x, g, b):
    m = jnp.mean(x, axis=-1, keepdims=True)
    v = jnp.mean((x - m) * (x - m), axis=-1, keepdims=True)
    return (x - m) * jax.lax.rsqrt(v + 1e-5) * g + b


def _post_body(x_ref, o_ref, wo_ref, bo_ref, g1_ref, b1_ref,
               w1_ref, bf1_ref, w2_ref, bf2_ref, g2_ref, b2_ref, out_ref):
    x = x_ref[...]
    y = x + _dot(o_ref[...], wo_ref[...]) + bo_ref[...]
    h1 = _ln(y, g1_ref[...], b1_ref[...])
    a = jax.nn.gelu(_dot(h1, w1_ref[...]) + bf1_ref[...])
    y2 = h1 + _dot(a, w2_ref[...]) + bf2_ref[...]
    out_ref[...] = _ln(y2, g2_ref[...], b2_ref[...])


def _post(x2, o2, p):
    n = _B * _L
    blk = 512
    vec = lambda a: a.reshape(1, -1)
    return pl.pallas_call(
        _post_body,
        grid=(n // blk,),
        in_specs=[
            pl.BlockSpec((blk, _D), lambda i: (i, 0)),
            pl.BlockSpec((blk, _D), lambda i: (i, 0)),
            pl.BlockSpec((_D, _D), lambda i: (0, 0)),
            pl.BlockSpec((1, _D), lambda i: (0, 0)),
            pl.BlockSpec((1, _D), lambda i: (0, 0)),
            pl.BlockSpec((1, _D), lambda i: (0, 0)),
            pl.BlockSpec((_D, _DFF), lambda i: (0, 0)),
            pl.BlockSpec((1, _DFF), lambda i: (0, 0)),
            pl.BlockSpec((_DFF, _D), lambda i: (0, 0)),
            pl.BlockSpec((1, _D), lambda i: (0, 0)),
            pl.BlockSpec((1, _D), lambda i: (0, 0)),
            pl.BlockSpec((1, _D), lambda i: (0, 0)),
        ],
        out_specs=pl.BlockSpec((blk, _D), lambda i: (i, 0)),
        out_shape=jax.ShapeDtypeStruct((n, _D), F32),
    )(x2, o2, p['Wo'], vec(p['bo']), vec(p['g1']), vec(p['b1']),
      p['W1'], vec(p['bf1']), p['W2'], vec(p['bf2']),
      vec(p['g2']), vec(p['b2']))


def _head_body(tgt_ref, xl_ref, tv_ref, gf_ref, bf_ref,
               wp_ref, bp_ref, out_ref):
    x = _ln(xl_ref[...], gf_ref[...], bf_ref[...])
    feat = jnp.concatenate([tv_ref[...], x], axis=1)
    score = _dot(feat, wp_ref[...]) + bp_ref[...]
    m = jnp.max(score, axis=1, keepdims=True)
    lse = m + jnp.log(jnp.sum(jnp.exp(score - m), axis=1, keepdims=True))
    row = jax.lax.broadcasted_iota(I32, (_B, 1), 0)
    tsel = jnp.where(row == 0, tgt_ref[0], tgt_ref[1])
    lane = jax.lax.broadcasted_iota(I32, (_B, _NPAD), 1)
    oh = (lane == tsel).astype(F32)
    st = jnp.sum(score * oh, axis=1, keepdims=True)
    out_ref[...] = -jnp.sum(st - lse, axis=0, keepdims=True) / _B


def _head(targets, x_last, tv_last, gf, bf, wp_pad, bp_pad):
    tvd = tv_last.shape[1]
    grid_spec = pltpu.PrefetchScalarGridSpec(
        num_scalar_prefetch=1,
        grid=(1,),
        in_specs=[
            pl.BlockSpec((_B, _D), lambda i, t: (0, 0)),
            pl.BlockSpec((_B, tvd), lambda i, t: (0, 0)),
            pl.BlockSpec((1, _D), lambda i, t: (0, 0)),
            pl.BlockSpec((1, _D), lambda i, t: (0, 0)),
            pl.BlockSpec((_D + tvd, _NPAD), lambda i, t: (0, 0)),
            pl.BlockSpec((1, _NPAD), lambda i, t: (0, 0)),
        ],
        out_specs=pl.BlockSpec((1, 1), lambda i, t: (0, 0)),
    )
    return pl.pallas_call(
        _head_body,
        grid_spec=grid_spec,
        out_shape=jax.ShapeDtypeStruct((1, 1), F32),
    )(targets, x_last, tv_last, gf, bf, wp_pad, bp_pad)


def kernel(x_app, x_time, time_vec, targets, mode, params):
    del mode
    vec = lambda a: a.reshape(1, -1)
    x = _embed(x_app[:, -1].astype(I32), x_time,
               params['Wt'], vec(params['bt']), params['app_table'])
    x2 = x.reshape(_B * _L, _D)
    for i, p in enumerate(params['layers']):
        items = _proj(x2.reshape(_B, _L, _D), p['Wqk'], p['Wv'])
        r_mat = jax.random.normal(
            jax.random.fold_in(jax.random.key(42), i),
            (_DH, _NHASH, _NB // 2), dtype=F32).reshape(_DH, -1)
        gd, bpc, bpr = _hash(items, r_mat)
        gd_flat = gd.transpose(0, 2, 1).reshape(-1)
        sorted_all = _sc_scatter(items.reshape(_B * _H * _L, _W_ITEM),
                                 gd_flat)
        sout = _attend(sorted_all.reshape(_B * _H, _NS, _W_ITEM), bpc, bpr)
        gat = _sc_gather(sout.reshape(_B * _H * _NS, _W_OUT), gd_flat)
        o2 = _combine(gat).reshape(_B * _L, _D)
        x2 = _post(x2, o2, p)
    x_last = x2.reshape(_B, _L, _D)[:, -1, :]
    tvd = time_vec.shape[-1]
    wp_pad = jnp.pad(params['Wp'], ((0, 0), (0, _NPAD - _NUM_APP)))
    bp_pad = jnp.pad(params['bp'], (0, _NPAD - _NUM_APP),
                     constant_values=-1e30)
    loss = _head(targets.astype(I32), x_last, time_vec[:, -1, :],
                 vec(params['gf']), vec(params['bf']), wp_pad, vec(bp_pad))
    return loss[0, 0]
